# Initial kernel scaffold; baseline (speedup 1.0000x reference)
import jax
import jax.numpy as jnp
from jax.experimental import pallas as pl


def kernel(x, batch, params):
    raise NotImplementedError("write your pallas kernel here")


# speedup vs baseline: 4.1925x; 4.1925x over previous
import functools

import jax
import jax.numpy as jnp
from jax import lax
from jax.experimental import pallas as pl
from jax.experimental.pallas import tpu as pltpu
from jax.experimental.pallas import tpu_sc as plsc

_K = 20
_EPS = 1e-5
_MASKED = 3e38


def _leaky(h):
    return jnp.where(h >= 0, h, 0.2 * h)


def _dot_bf16(a, b):
    return lax.dot_general(a.astype(jnp.bfloat16), b.astype(jnp.bfloat16),
                           (((1,), (0,)), ((), ())),
                           preferred_element_type=jnp.float32)


def _knn_body(x_ref, xt_ref, bcand_ref, bquery_ref, out_ref, d_scr, *,
              n, qb, rc, k):
    j = pl.program_id(0)
    nch = n // rc
    xb = xt_ref[...]
    sqb = jnp.sum(xb * xb, axis=0, keepdims=True)
    bquery = bquery_ref[...]
    col0 = j * qb

    def init_body(rcI, _):
        off = pl.multiple_of(rcI * rc, rc)
        xc = x_ref[pl.ds(off, rc), :]
        sqr = jnp.sum(xc * xc, axis=1, keepdims=True)
        prod = lax.dot_general(xc, xb, (((1,), (0,)), ((), ())),
                               preferred_element_type=jnp.float32)
        d = sqr + sqb - 2.0 * prod
        bcand = bcand_ref[pl.ds(off, rc), :]
        rid = off + lax.broadcasted_iota(jnp.int32, (rc, qb), 0)
        cid = col0 + lax.broadcasted_iota(jnp.int32, (rc, qb), 1)
        d = jnp.where(bcand != bquery, _MASKED, d)
        d = jnp.where(rid == cid, _MASKED, d)
        d_scr[pl.ds(off, rc), :] = d
        return 0

    lax.fori_loop(0, nch, init_body, 0)

    prev = jnp.full((1, qb), -1, jnp.int32)
    for t in range(k):
        def sel_body(rcI, carry, prev=prev):
            m, ix = carry
            off = pl.multiple_of(rcI * rc, rc)
            d = d_scr[pl.ds(off, rc), :]
            rid = off + lax.broadcasted_iota(jnp.int32, (rc, qb), 0)
            d = jnp.where(rid == prev, jnp.inf, d)
            d_scr[pl.ds(off, rc), :] = d
            cm = jnp.min(d, axis=0, keepdims=True)
            cix = jnp.min(jnp.where(d == cm, rid, n), axis=0, keepdims=True)
            take = (cm < m) | ((cm == m) & (cix < ix))
            return (jnp.where(take, cm, m), jnp.where(take, cix, ix))

        m0 = jnp.full((1, qb), jnp.inf, jnp.float32)
        ix0 = jnp.full((1, qb), n, jnp.int32)
        _, ix = lax.fori_loop(0, nch, sel_body, (m0, ix0))
        out_ref[pl.ds(t, 1), :] = ix
        prev = ix


def _knn_neighbors(x, batch):
    n, c = x.shape
    qb = 256
    rc = 512
    kp = 24
    out = pl.pallas_call(
        functools.partial(_knn_body, n=n, qb=qb, rc=rc, k=_K),
        grid=(n // qb,),
        in_specs=[
            pl.BlockSpec((n, c), lambda j: (0, 0)),
            pl.BlockSpec((c, qb), lambda j: (0, j)),
            pl.BlockSpec((n, 1), lambda j: (0, 0)),
            pl.BlockSpec((1, qb), lambda j: (0, j)),
        ],
        out_specs=pl.BlockSpec((kp, qb), lambda j: (0, j)),
        out_shape=jax.ShapeDtypeStruct((kp, n), jnp.int32),
        scratch_shapes=[pltpu.VMEM((n, qb), jnp.float32)],
    )(x, x.T, batch.reshape(n, 1), batch.reshape(1, n))
    return out[:_K]


def _sc_gather(table, idx_flat):
    v, c = table.shape
    e = idx_flat.shape[0]
    info = plsc.get_sparse_core_info()
    nw = info.num_cores * info.num_subcores
    rw = e // nw
    ch = 128
    nchunk = rw // ch
    mesh = plsc.VectorSubcoreMesh(core_axis_name="c", subcore_axis_name="s")

    @functools.partial(
        pl.kernel, mesh=mesh,
        out_type=jax.ShapeDtypeStruct((e, c), jnp.float32),
        scratch_types=[
            pltpu.VMEM((ch,), jnp.int32),
            pltpu.VMEM((ch, c), jnp.float32),
            pltpu.SemaphoreType.DMA,
        ],
    )
    def gk(table_hbm, idx_hbm, out_hbm, idx_v, rows_v, sem):
        wid = lax.axis_index("s") * info.num_cores + lax.axis_index("c")
        base0 = wid * rw

        def body(ci, _):
            base = base0 + ci * ch
            pltpu.sync_copy(idx_hbm.at[pl.ds(base, ch)], idx_v)
            pltpu.async_copy(table_hbm.at[idx_v], rows_v, sem).wait()
            pltpu.sync_copy(rows_v, out_hbm.at[pl.ds(base, ch)])
            return 0

        lax.fori_loop(0, nchunk, body, 0)

    return gk(table, idx_flat)


def _affine_from_stats(st, g, be, e_count):
    mu = st[0:1, :] / e_count
    var = st[1:2, :] / e_count - mu * mu
    alpha = g * lax.rsqrt(var + _EPS)
    beta = be - alpha * mu
    return alpha, beta


def _prep_body(tin_ref, w1_ref, b1_ref, u_ref, tp_ref, *, cin, cout):
    t = tin_ref[...]
    w1a = w1_ref[0:cin, :]
    u_ref[...] = _dot_bf16(t, w1a) + b1_ref[...]
    tp_ref[:, 0:cin] = t


def _prep_affine_body(m_ref, st_ref, g_ref, be_ref, w1_ref, b1_ref,
                      x_ref, u_ref, tp_ref, *, cin, cout, e_count):
    alpha, beta = _affine_from_stats(st_ref[...], g_ref[...], be_ref[...],
                                     e_count)
    t = _leaky(alpha * m_ref[...] + beta)
    x_ref[...] = t
    w1a = w1_ref[0:cin, :]
    u_ref[...] = _dot_bf16(t, w1a) + b1_ref[...]
    tp_ref[:, 0:cin] = t


def _prep(t_in, w1, b1):
    n, cin = t_in.shape
    cout = w1.shape[1]
    cpad = ((cin + 127) // 128) * 128
    rb = 512
    return pl.pallas_call(
        functools.partial(_prep_body, cin=cin, cout=cout),
        grid=(n // rb,),
        in_specs=[
            pl.BlockSpec((rb, cin), lambda i: (i, 0)),
            pl.BlockSpec((2 * cin, cout), lambda i: (0, 0)),
            pl.BlockSpec((1, cout), lambda i: (0, 0)),
        ],
        out_specs=[
            pl.BlockSpec((rb, cout), lambda i: (i, 0)),
            pl.BlockSpec((rb, cpad), lambda i: (i, 0)),
        ],
        out_shape=[
            jax.ShapeDtypeStruct((n, cout), jnp.float32),
            jax.ShapeDtypeStruct((n, cpad), jnp.float32),
        ],
    )(t_in, w1, b1.reshape(1, cout))


def _prep_affine(m_prev, stats_prev, g, be, w1, b1, e_count):
    n, cin = m_prev.shape
    cout = w1.shape[1]
    cpad = ((cin + 127) // 128) * 128
    rb = 512
    return pl.pallas_call(
        functools.partial(_prep_affine_body, cin=cin, cout=cout,
                          e_count=e_count),
        grid=(n // rb,),
        in_specs=[
            pl.BlockSpec((rb, cin), lambda i: (i, 0)),
            pl.BlockSpec((8, cin), lambda i: (0, 0)),
            pl.BlockSpec((1, cin), lambda i: (0, 0)),
            pl.BlockSpec((1, cin), lambda i: (0, 0)),
            pl.BlockSpec((2 * cin, cout), lambda i: (0, 0)),
            pl.BlockSpec((1, cout), lambda i: (0, 0)),
        ],
        out_specs=[
            pl.BlockSpec((rb, cin), lambda i: (i, 0)),
            pl.BlockSpec((rb, cout), lambda i: (i, 0)),
            pl.BlockSpec((rb, cpad), lambda i: (i, 0)),
        ],
        out_shape=[
            jax.ShapeDtypeStruct((n, cin), jnp.float32),
            jax.ShapeDtypeStruct((n, cout), jnp.float32),
            jax.ShapeDtypeStruct((n, cpad), jnp.float32),
        ],
    )(m_prev, stats_prev, g.reshape(1, cin), be.reshape(1, cin),
      w1, b1.reshape(1, cout))


def _stats1_body(xg_ref, xi_ref, u_ref, w1_ref, out_ref):
    k = pl.program_id(0)
    nb = pl.program_id(1)
    cin = xi_ref.shape[1]
    e = xg_ref[:, 0:cin] - xi_ref[...]
    z = u_ref[...] + _dot_bf16(e, w1_ref[...])
    s = jnp.sum(z, axis=0, keepdims=True)
    q = jnp.sum(z * z, axis=0, keepdims=True)
    upd = jnp.concatenate(
        [s, q, jnp.zeros((6, z.shape[1]), jnp.float32)], axis=0)

    @pl.when(jnp.logical_and(k == 0, nb == 0))
    def _():
        out_ref[...] = upd

    @pl.when(jnp.logical_not(jnp.logical_and(k == 0, nb == 0)))
    def _():
        out_ref[...] = out_ref[...] + upd


def _stats1(xg, x_l, u, w1b):
    n, cin = x_l.shape
    cout = w1b.shape[1]
    cpad = xg.shape[1]
    rb = 512
    nbn = n // rb
    return pl.pallas_call(
        _stats1_body,
        grid=(_K, nbn),
        in_specs=[
            pl.BlockSpec((rb, cpad), lambda k, i: (k * nbn + i, 0)),
            pl.BlockSpec((rb, cin), lambda k, i: (i, 0)),
            pl.BlockSpec((rb, cout), lambda k, i: (i, 0)),
            pl.BlockSpec((cin, cout), lambda k, i: (0, 0)),
        ],
        out_specs=pl.BlockSpec((8, cout), lambda k, i: (0, 0)),
        out_shape=jax.ShapeDtypeStruct((8, cout), jnp.float32),
    )(xg, x_l, u, w1b)


def _convout_body(xg_ref, xi_ref, u_ref, w1_ref, st1_ref, g1_ref,
                  be1_ref, w2_ref, b2_ref, m_ref, st2_ref, *, e_count):
    nb = pl.program_id(0)
    k = pl.program_id(1)
    alpha, beta = _affine_from_stats(st1_ref[...], g1_ref[...], be1_ref[...],
                                     e_count)
    cin = xi_ref.shape[1]
    e = xg_ref[:, 0:cin] - xi_ref[...]
    z1 = u_ref[...] + _dot_bf16(e, w1_ref[...])
    a1 = _leaky(alpha * z1 + beta)
    z2 = _dot_bf16(a1, w2_ref[...]) + b2_ref[...]
    s = jnp.sum(z2, axis=0, keepdims=True)
    q = jnp.sum(z2 * z2, axis=0, keepdims=True)
    upd = jnp.concatenate(
        [s, q, jnp.zeros((6, z2.shape[1]), jnp.float32)], axis=0)

    @pl.when(jnp.logical_and(k == 0, nb == 0))
    def _():
        st2_ref[...] = upd

    @pl.when(jnp.logical_not(jnp.logical_and(k == 0, nb == 0)))
    def _():
        st2_ref[...] = st2_ref[...] + upd

    @pl.when(k == 0)
    def _():
        m_ref[...] = z2

    @pl.when(k > 0)
    def _():
        m_ref[...] = jnp.maximum(m_ref[...], z2)


def _convout(xg, x_l, u, w1b, stats1, g1, be1, w2, b2, e_count):
    n, cin = x_l.shape
    c = w1b.shape[1]
    cpad = xg.shape[1]
    cout = w2.shape[1]
    rb = 512
    nbn = n // rb
    return pl.pallas_call(
        functools.partial(_convout_body, e_count=e_count),
        grid=(nbn, _K),
        in_specs=[
            pl.BlockSpec((rb, cpad), lambda i, k: (k * nbn + i, 0)),
            pl.BlockSpec((rb, cin), lambda i, k: (i, 0)),
            pl.BlockSpec((rb, c), lambda i, k: (i, 0)),
            pl.BlockSpec((cin, c), lambda i, k: (0, 0)),
            pl.BlockSpec((8, c), lambda i, k: (0, 0)),
            pl.BlockSpec((1, c), lambda i, k: (0, 0)),
            pl.BlockSpec((1, c), lambda i, k: (0, 0)),
            pl.BlockSpec((c, cout), lambda i, k: (0, 0)),
            pl.BlockSpec((1, cout), lambda i, k: (0, 0)),
        ],
        out_specs=[
            pl.BlockSpec((rb, cout), lambda i, k: (i, 0)),
            pl.BlockSpec((8, cout), lambda i, k: (0, 0)),
        ],
        out_shape=[
            jax.ShapeDtypeStruct((n, cout), jnp.float32),
            jax.ShapeDtypeStruct((8, cout), jnp.float32),
        ],
    )(xg, x_l, u, w1b, stats1, g1.reshape(1, c), be1.reshape(1, c),
      w2, b2.reshape(1, cout))


def _head_body(x1_ref, x2_ref, x3_ref, x4_ref, m5_ref, st5_ref, g5_ref,
               be5_ref, wa_ref, ba_ref, wb_ref, bb_ref, wc_ref, bc_ref,
               out_ref, *, e_count, splits):
    alpha, beta = _affine_from_stats(st5_ref[...], g5_ref[...], be5_ref[...],
                                     e_count)
    x5 = _leaky(alpha * m5_ref[...] + beta)
    xs = [x1_ref[...], x2_ref[...], x3_ref[...], x4_ref[...], x5]
    acc = ba_ref[...]
    off = 0
    for xpart in xs:
        cw = xpart.shape[1]
        acc = acc + _dot_bf16(xpart, wa_ref[off:off + cw, :])
        off += cw
    h = jnp.maximum(acc, 0.0)
    h = jnp.maximum(_dot_bf16(h, wb_ref[...]) + bb_ref[...], 0.0)
    out_ref[...] = _dot_bf16(h, wc_ref[...]) + bc_ref[...]


def _head(xs, m5, stats5, g5, be5, hp, e_count):
    n = m5.shape[0]
    rb = 512
    c1, c2, c3, c4 = (x.shape[1] for x in xs)
    c5 = m5.shape[1]
    ctot = c1 + c2 + c3 + c4 + c5
    ha, hb, hc = hp["Wa"].shape[1], hp["Wb"].shape[1], hp["Wc"].shape[1]
    return pl.pallas_call(
        functools.partial(_head_body, e_count=e_count,
                          splits=(c1, c2, c3, c4, c5)),
        grid=(n // rb,),
        in_specs=[
            pl.BlockSpec((rb, c1), lambda i: (i, 0)),
            pl.BlockSpec((rb, c2), lambda i: (i, 0)),
            pl.BlockSpec((rb, c3), lambda i: (i, 0)),
            pl.BlockSpec((rb, c4), lambda i: (i, 0)),
            pl.BlockSpec((rb, c5), lambda i: (i, 0)),
            pl.BlockSpec((8, c5), lambda i: (0, 0)),
            pl.BlockSpec((1, c5), lambda i: (0, 0)),
            pl.BlockSpec((1, c5), lambda i: (0, 0)),
            pl.BlockSpec((ctot, ha), lambda i: (0, 0)),
            pl.BlockSpec((1, ha), lambda i: (0, 0)),
            pl.BlockSpec((ha, hb), lambda i: (0, 0)),
            pl.BlockSpec((1, hb), lambda i: (0, 0)),
            pl.BlockSpec((hb, hc), lambda i: (0, 0)),
            pl.BlockSpec((1, hc), lambda i: (0, 0)),
        ],
        out_specs=pl.BlockSpec((rb, hc), lambda i: (i, 0)),
        out_shape=jax.ShapeDtypeStruct((n, hc), jnp.float32),
    )(xs[0], xs[1], xs[2], xs[3], m5, stats5,
      g5.reshape(1, c5), be5.reshape(1, c5),
      hp["Wa"], hp["ba"].reshape(1, ha),
      hp["Wb"], hp["bb"].reshape(1, hb),
      hp["Wc"], hp["bc"].reshape(1, hc))


def kernel(x, batch, params):
    n = x.shape[0]
    e_count = float(n * _K)
    nbr_t = _knn_neighbors(x, batch.astype(jnp.int32))
    idx_flat = nbr_t.reshape(-1)

    convs = [params["conv1"], params["conv2"], params["conv3"],
             params["conv4"], params["conv5"]]
    xs = []
    t_in = x
    m = None
    stats2 = None
    prev = None
    x_l = t_in
    for p in convs:
        if prev is None:
            u, tp = _prep(t_in, p["W1"], p["b1"])
        else:
            x_l, u, tp = _prep_affine(m, stats2, prev["g2"], prev["be2"],
                                      p["W1"], p["b1"], e_count)
            xs.append(x_l)
        w1b = p["W1"][x_l.shape[1]:]
        xg = _sc_gather(tp, idx_flat)
        st1 = _stats1(xg, x_l, u, w1b)
        m, stats2 = _convout(xg, x_l, u, w1b, st1, p["g1"], p["be1"],
                             p["W2"], p["b2"], e_count)
        prev = p

    return _head(xs, m, stats2, prev["g2"], prev["be2"], params["head"],
                 e_count)
